# baseline (speedup 1.0000x reference)
import functools

import jax
import jax.numpy as jnp
from jax import lax
from jax.experimental import pallas as pl
from jax.experimental.pallas import tpu as pltpu
from jax.experimental.pallas import tpu_sc as plsc

BATCH = 16384
EMBED = 32
PACK = 128 // EMBED


def _make_gather():
  info = plsc.get_sparse_core_info()
  nc, ns = info.num_cores, info.num_subcores
  nw = nc * ns
  b_per_w = BATCH // nw

  mesh = plsc.VectorSubcoreMesh(core_axis_name="c", subcore_axis_name="s")

  @functools.partial(
      pl.kernel,
      mesh=mesh,
      out_type=[
          jax.ShapeDtypeStruct((BATCH * EMBED,), jnp.float32),
          jax.ShapeDtypeStruct((BATCH * EMBED,), jnp.float32),
      ],
      scratch_types=[
          pltpu.VMEM((b_per_w,), jnp.int32),
          pltpu.VMEM((b_per_w,), jnp.int32),
          pltpu.VMEM((b_per_w * EMBED,), jnp.float32),
          pltpu.VMEM((b_per_w * EMBED,), jnp.float32),
          pltpu.SemaphoreType.DMA,
      ],
  )
  def gather(uidx_hbm, midx_hbm, utab_hbm, mtab_hbm, uout_hbm, mout_hbm,
             uidx_v, midx_v, ubuf_v, mbuf_v, sem):
    wid = lax.axis_index("s") * nc + lax.axis_index("c")
    base = wid * b_per_w
    pltpu.sync_copy(uidx_hbm.at[pl.ds(base, b_per_w)], uidx_v)
    pltpu.sync_copy(midx_hbm.at[pl.ds(base, b_per_w)], midx_v)

    def issue(g, carry):
      uvec = uidx_v[pl.ds(g * 16, 16)] * EMBED
      mvec = midx_v[pl.ds(g * 16, 16)] * EMBED
      for k in range(16):
        off = pl.multiple_of(uvec[k], EMBED)
        pltpu.async_copy(utab_hbm.at[pl.ds(off, EMBED)],
                         ubuf_v.at[pl.ds((g * 16 + k) * EMBED, EMBED)], sem)
        off2 = pl.multiple_of(mvec[k], EMBED)
        pltpu.async_copy(mtab_hbm.at[pl.ds(off2, EMBED)],
                         mbuf_v.at[pl.ds((g * 16 + k) * EMBED, EMBED)], sem)
      return carry

    lax.fori_loop(0, b_per_w // 16, issue, 0)
    pltpu.make_async_copy(uout_hbm.at[pl.ds(0, b_per_w * EMBED)], ubuf_v,
                          sem).wait()
    pltpu.make_async_copy(mout_hbm.at[pl.ds(0, b_per_w * EMBED)], mbuf_v,
                          sem).wait()
    pltpu.sync_copy(ubuf_v, uout_hbm.at[pl.ds(base * EMBED, b_per_w * EMBED)])
    pltpu.sync_copy(mbuf_v, mout_hbm.at[pl.ds(base * EMBED, b_per_w * EMBED)])

  return gather


def _mlp_body(u_ref, m_ref, w1u_ref, w1m_ref, b1_ref, w2_ref, b2_ref,
              w3_ref, b3_ref, o_ref):
  h1 = jnp.dot(u_ref[...], w1u_ref[...], preferred_element_type=jnp.float32)
  h1 = h1 + jnp.dot(m_ref[...], w1m_ref[...],
                    preferred_element_type=jnp.float32)
  h1 = jnp.maximum(h1 + b1_ref[...], 0.0)
  h2 = jnp.dot(h1, w2_ref[...], preferred_element_type=jnp.float32)
  h2 = jnp.maximum(h2 + b2_ref[...], 0.0)
  o_ref[...] = jnp.dot(h2, w3_ref[...],
                       preferred_element_type=jnp.float32) + b3_ref[...]


def _mlp_call(u_pack, m_pack, W1, b1, W2, b2, W3, b3):
  rows = BATCH // PACK
  br = 512
  grid = (rows // br,)
  eye = jnp.eye(PACK, dtype=jnp.float32)
  w1u = jnp.kron(eye, W1[:EMBED, :])
  w1m = jnp.kron(eye, W1[EMBED:, :])
  w2 = jnp.kron(eye, W2)
  w3 = jnp.kron(eye, W3)
  b1t = jnp.tile(b1, PACK).reshape(1, PACK * 128)
  b2t = jnp.tile(b2, PACK).reshape(1, PACK * 64)
  b3t = jnp.tile(b3, PACK).reshape(1, PACK)
  return pl.pallas_call(
      _mlp_body,
      grid=grid,
      in_specs=[
          pl.BlockSpec((br, 128), lambda i: (i, 0)),
          pl.BlockSpec((br, 128), lambda i: (i, 0)),
          pl.BlockSpec((128, PACK * 128), lambda i: (0, 0)),
          pl.BlockSpec((128, PACK * 128), lambda i: (0, 0)),
          pl.BlockSpec((1, PACK * 128), lambda i: (0, 0)),
          pl.BlockSpec((PACK * 128, PACK * 64), lambda i: (0, 0)),
          pl.BlockSpec((1, PACK * 64), lambda i: (0, 0)),
          pl.BlockSpec((PACK * 64, PACK), lambda i: (0, 0)),
          pl.BlockSpec((1, PACK), lambda i: (0, 0)),
      ],
      out_specs=pl.BlockSpec((br, PACK), lambda i: (i, 0)),
      out_shape=jax.ShapeDtypeStruct((rows, PACK), jnp.float32),
  )(u_pack, m_pack, w1u, w1m, b1t, w2, b2t, w3, b3t)


def kernel(user_input, movie_input, user_table, movie_table,
           W1, b1, W2, b2, W3, b3):
  gather = _make_gather()
  u_flat, m_flat = gather(user_input, movie_input,
                          user_table.reshape(-1), movie_table.reshape(-1))
  u_pack = u_flat.reshape(BATCH // PACK, 128)
  m_pack = m_flat.reshape(BATCH // PACK, 128)
  out = _mlp_call(u_pack, m_pack, W1, b1, W2, b2, W3, b3)
  return out.reshape(BATCH, 1)


# speedup vs baseline: 1.0019x; 1.0019x over previous
import functools

import jax
import jax.numpy as jnp
from jax import lax
from jax.experimental import pallas as pl
from jax.experimental.pallas import tpu as pltpu
from jax.experimental.pallas import tpu_sc as plsc

BATCH = 16384
EMBED = 32
CHUNK = 128
PACK = 128 // EMBED


def _make_gather():
  info = plsc.get_sparse_core_info()
  nc, ns = info.num_cores, info.num_subcores
  nw = nc * ns
  b_per_w = BATCH // nw
  n_chunks = b_per_w // CHUNK

  mesh = plsc.VectorSubcoreMesh(core_axis_name="c", subcore_axis_name="s")

  @functools.partial(
      pl.kernel,
      mesh=mesh,
      compiler_params=pltpu.CompilerParams(use_tc_tiling_on_sc=False),
      out_type=[
          jax.ShapeDtypeStruct((BATCH, EMBED), jnp.float32),
          jax.ShapeDtypeStruct((BATCH, EMBED), jnp.float32),
      ],
      scratch_types=[
          pltpu.VMEM((n_chunks, CHUNK), jnp.int32),
          pltpu.VMEM((n_chunks, CHUNK), jnp.int32),
          pltpu.VMEM((b_per_w, EMBED), jnp.float32),
          pltpu.VMEM((b_per_w, EMBED), jnp.float32),
          pltpu.SemaphoreType.DMA,
      ],
  )
  def gather(uidx_hbm, midx_hbm, utab_hbm, mtab_hbm, uout_hbm, mout_hbm,
             uidx_v, midx_v, urows_v, mrows_v, sem):
    wid = lax.axis_index("s") * nc + lax.axis_index("c")
    base = wid * b_per_w
    crow = wid * n_chunks
    pltpu.sync_copy(uidx_hbm.at[pl.ds(crow, n_chunks)], uidx_v)
    pltpu.sync_copy(midx_hbm.at[pl.ds(crow, n_chunks)], midx_v)
    copies = []
    for j in range(n_chunks):
      copies.append(pltpu.async_copy(
          utab_hbm.at[uidx_v.at[j]],
          urows_v.at[pl.ds(j * CHUNK, CHUNK)], sem))
      copies.append(pltpu.async_copy(
          mtab_hbm.at[midx_v.at[j]],
          mrows_v.at[pl.ds(j * CHUNK, CHUNK)], sem))
    for c in copies:
      c.wait()
    pltpu.sync_copy(urows_v, uout_hbm.at[pl.ds(base, b_per_w)])
    pltpu.sync_copy(mrows_v, mout_hbm.at[pl.ds(base, b_per_w)])

  return gather


def _mlp_body(u_ref, m_ref, w1u_ref, w1m_ref, b1_ref, w2_ref, b2_ref,
              w3_ref, b3_ref, o_ref):
  h1 = jnp.dot(u_ref[...], w1u_ref[...], preferred_element_type=jnp.float32)
  h1 = h1 + jnp.dot(m_ref[...], w1m_ref[...],
                    preferred_element_type=jnp.float32)
  h1 = jnp.maximum(h1 + b1_ref[...], 0.0)
  h2 = jnp.dot(h1, w2_ref[...], preferred_element_type=jnp.float32)
  h2 = jnp.maximum(h2 + b2_ref[...], 0.0)
  o_ref[...] = jnp.dot(h2, w3_ref[...],
                       preferred_element_type=jnp.float32) + b3_ref[...]


def _mlp_call(u_pack, m_pack, W1, b1, W2, b2, W3, b3):
  rows = BATCH // PACK
  br = 512
  grid = (rows // br,)
  eye = jnp.eye(PACK, dtype=jnp.float32)
  w1u = jnp.kron(eye, W1[:EMBED, :])
  w1m = jnp.kron(eye, W1[EMBED:, :])
  w2 = jnp.kron(eye, W2)
  w3 = jnp.kron(eye, W3)
  b1t = jnp.tile(b1, PACK).reshape(1, PACK * 128)
  b2t = jnp.tile(b2, PACK).reshape(1, PACK * 64)
  b3t = jnp.tile(b3, PACK).reshape(1, PACK)
  return pl.pallas_call(
      _mlp_body,
      grid=grid,
      in_specs=[
          pl.BlockSpec((br, 128), lambda i: (i, 0)),
          pl.BlockSpec((br, 128), lambda i: (i, 0)),
          pl.BlockSpec((128, PACK * 128), lambda i: (0, 0)),
          pl.BlockSpec((128, PACK * 128), lambda i: (0, 0)),
          pl.BlockSpec((1, PACK * 128), lambda i: (0, 0)),
          pl.BlockSpec((PACK * 128, PACK * 64), lambda i: (0, 0)),
          pl.BlockSpec((1, PACK * 64), lambda i: (0, 0)),
          pl.BlockSpec((PACK * 64, PACK), lambda i: (0, 0)),
          pl.BlockSpec((1, PACK), lambda i: (0, 0)),
      ],
      out_specs=pl.BlockSpec((br, PACK), lambda i: (i, 0)),
      out_shape=jax.ShapeDtypeStruct((rows, PACK), jnp.float32),
  )(u_pack, m_pack, w1u, w1m, b1t, w2, b2t, w3, b3t)


def kernel(user_input, movie_input, user_table, movie_table,
           W1, b1, W2, b2, W3, b3):
  gather = _make_gather()
  uidx2 = user_input.reshape(BATCH // CHUNK, CHUNK)
  midx2 = movie_input.reshape(BATCH // CHUNK, CHUNK)
  u_emb, m_emb = gather(uidx2, midx2, user_table, movie_table)
  u_pack = u_emb.reshape(BATCH // PACK, 128)
  m_pack = m_emb.reshape(BATCH // PACK, 128)
  out = _mlp_call(u_pack, m_pack, W1, b1, W2, b2, W3, b3)
  return out.reshape(BATCH, 1)


# speedup vs baseline: 1.4687x; 1.4660x over previous
import functools

import jax
import jax.numpy as jnp
from jax import lax
from jax.experimental import pallas as pl
from jax.experimental.pallas import tpu as pltpu
from jax.experimental.pallas import tpu_sc as plsc

BATCH = 16384
EMBED = 32
GROUP = 8
CH = 32


def _transpose_body(t_ref, o_ref):
  o_ref[...] = t_ref[...].T


def _transpose_call(tab_t):
  n = tab_t.shape[1]
  cb = 8192
  grid = (pl.cdiv(n, cb),)
  return pl.pallas_call(
      _transpose_body,
      grid=grid,
      in_specs=[pl.BlockSpec((EMBED, cb), lambda i: (0, i))],
      out_specs=pl.BlockSpec((cb, EMBED), lambda i: (i, 0)),
      out_shape=jax.ShapeDtypeStruct((n, EMBED), jnp.float32),
  )(tab_t)


def _make_gather():
  info = plsc.get_sparse_core_info()
  nc, ns = info.num_cores, info.num_subcores
  nw = nc * ns
  b_per_w = BATCH // nw
  n_ch = b_per_w // CH

  mesh = plsc.VectorSubcoreMesh(core_axis_name="c", subcore_axis_name="s")

  @functools.partial(
      pl.kernel,
      mesh=mesh,
      out_type=[
          jax.ShapeDtypeStruct((BATCH, EMBED), jnp.float32),
          jax.ShapeDtypeStruct((BATCH, EMBED), jnp.float32),
      ],
      scratch_types=[
          pltpu.VMEM((b_per_w,), jnp.int32),
          pltpu.VMEM((b_per_w,), jnp.int32),
          pltpu.VMEM((CH * GROUP, EMBED), jnp.float32),
          pltpu.VMEM((CH * GROUP, EMBED), jnp.float32),
          pltpu.VMEM((CH, EMBED), jnp.float32),
          pltpu.VMEM((CH, EMBED), jnp.float32),
          pltpu.SemaphoreType.DMA,
      ],
  )
  def gather(uidx_hbm, midx_hbm, utab_hbm, mtab_hbm, uout_hbm, mout_hbm,
             uidx_v, midx_v, ubuf_v, mbuf_v, uo_v, mo_v, sem):
    wid = lax.axis_index("s") * nc + lax.axis_index("c")
    base = wid * b_per_w
    pltpu.sync_copy(uidx_hbm.at[pl.ds(base, b_per_w)], uidx_v)
    pltpu.sync_copy(midx_hbm.at[pl.ds(base, b_per_w)], midx_v)

    for c in range(n_ch):
      def issue(g, carry):
        uvec = (uidx_v[pl.ds(c * CH + g * 16, 16)] >> 3) * GROUP
        mvec = (midx_v[pl.ds(c * CH + g * 16, 16)] >> 3) * GROUP
        for k in range(16):
          rr = pl.multiple_of(uvec[k], GROUP)
          pltpu.async_copy(utab_hbm.at[pl.ds(rr, GROUP)],
                           ubuf_v.at[pl.ds((g * 16 + k) * GROUP, GROUP)], sem)
          ss = pl.multiple_of(mvec[k], GROUP)
          pltpu.async_copy(mtab_hbm.at[pl.ds(ss, GROUP)],
                           mbuf_v.at[pl.ds((g * 16 + k) * GROUP, GROUP)], sem)
        return carry

      lax.fori_loop(0, CH // 16, issue, 0)
      pltpu.make_async_copy(uout_hbm.at[pl.ds(0, CH * GROUP)], ubuf_v,
                            sem).wait()
      pltpu.make_async_copy(mout_hbm.at[pl.ds(0, CH * GROUP)], mbuf_v,
                            sem).wait()

      def select(g, carry):
        uq = (uidx_v[pl.ds(c * CH + g * 16, 16)] & (GROUP - 1))
        mq = (midx_v[pl.ds(c * CH + g * 16, 16)] & (GROUP - 1))
        for k in range(16):
          i = g * 16 + k
          ur = i * GROUP + uq[k]
          uo_v[i, pl.ds(0, 16)] = ubuf_v[ur, pl.ds(0, 16)]
          uo_v[i, pl.ds(16, 16)] = ubuf_v[ur, pl.ds(16, 16)]
          mr = i * GROUP + mq[k]
          mo_v[i, pl.ds(0, 16)] = mbuf_v[mr, pl.ds(0, 16)]
          mo_v[i, pl.ds(16, 16)] = mbuf_v[mr, pl.ds(16, 16)]
        return carry

      lax.fori_loop(0, CH // 16, select, 0)
      pltpu.sync_copy(uo_v, uout_hbm.at[pl.ds(base + c * CH, CH)])
      pltpu.sync_copy(mo_v, mout_hbm.at[pl.ds(base + c * CH, CH)])

  return gather


def _mlp_body(u_ref, m_ref, w1_ref, b1_ref, w2_ref, b2_ref, w3_ref, b3_ref,
              o_ref):
  h1 = jnp.dot(u_ref[...], w1_ref[0:EMBED, :],
               preferred_element_type=jnp.float32)
  h1 = h1 + jnp.dot(m_ref[...], w1_ref[EMBED:2 * EMBED, :],
                    preferred_element_type=jnp.float32)
  h1 = jnp.maximum(h1 + b1_ref[...], 0.0)
  h2 = jnp.dot(h1, w2_ref[...], preferred_element_type=jnp.float32)
  h2 = jnp.maximum(h2 + b2_ref[...], 0.0)
  o_ref[...] = jnp.sum(h2 * w3_ref[...], axis=1, keepdims=True) + b3_ref[...]


def _mlp_call(u_emb, m_emb, W1, b1, W2, b2, W3, b3):
  bb = 2048
  grid = (BATCH // bb,)
  return pl.pallas_call(
      _mlp_body,
      grid=grid,
      in_specs=[
          pl.BlockSpec((bb, EMBED), lambda i: (i, 0)),
          pl.BlockSpec((bb, EMBED), lambda i: (i, 0)),
          pl.BlockSpec((2 * EMBED, 128), lambda i: (0, 0)),
          pl.BlockSpec((1, 128), lambda i: (0, 0)),
          pl.BlockSpec((128, 64), lambda i: (0, 0)),
          pl.BlockSpec((1, 64), lambda i: (0, 0)),
          pl.BlockSpec((1, 64), lambda i: (0, 0)),
          pl.BlockSpec((1, 1), lambda i: (0, 0)),
      ],
      out_specs=pl.BlockSpec((bb, 1), lambda i: (i, 0)),
      out_shape=jax.ShapeDtypeStruct((BATCH, 1), jnp.float32),
  )(u_emb, m_emb, W1, b1.reshape(1, 128), W2, b2.reshape(1, 64),
    W3.reshape(1, 64), b3.reshape(1, 1))


def kernel(user_input, movie_input, user_table, movie_table,
           W1, b1, W2, b2, W3, b3):
  utab = _transpose_call(user_table.T)
  mtab = _transpose_call(movie_table.T)
  gather = _make_gather()
  u_emb, m_emb = gather(user_input, movie_input, utab, mtab)
  return _mlp_call(u_emb, m_emb, W1, b1, W2, b2, W3, b3)


# speedup vs baseline: 1.6318x; 1.1111x over previous
import functools

import jax
import jax.numpy as jnp
from jax import lax
from jax.experimental import pallas as pl
from jax.experimental.pallas import tpu as pltpu
from jax.experimental.pallas import tpu_sc as plsc

BATCH = 16384
EMBED = 32
GROUP = 8
CH = 32


def _transpose_body(t_ref, o_ref):
  o_ref[...] = t_ref[...].T


def _transpose_call(tab_t):
  n = tab_t.shape[1]
  cb = 32768
  grid = (pl.cdiv(n, cb),)
  return pl.pallas_call(
      _transpose_body,
      grid=grid,
      in_specs=[pl.BlockSpec((EMBED, cb), lambda i: (0, i))],
      out_specs=pl.BlockSpec((cb, EMBED), lambda i: (i, 0)),
      out_shape=jax.ShapeDtypeStruct((n, EMBED), jnp.float32),
  )(tab_t)


def _make_gather():
  info = plsc.get_sparse_core_info()
  nc, ns = info.num_cores, info.num_subcores
  nw = nc * ns
  b_per_w = BATCH // nw
  n_ch = b_per_w // CH

  mesh = plsc.VectorSubcoreMesh(core_axis_name="c", subcore_axis_name="s")

  @functools.partial(
      pl.kernel,
      mesh=mesh,
      out_type=[
          jax.ShapeDtypeStruct((BATCH, EMBED), jnp.float32),
          jax.ShapeDtypeStruct((BATCH, EMBED), jnp.float32),
      ],
      scratch_types=[
          pltpu.VMEM((b_per_w,), jnp.int32),
          pltpu.VMEM((b_per_w,), jnp.int32),
          pltpu.VMEM((CH * GROUP, EMBED), jnp.float32),
          pltpu.VMEM((CH * GROUP, EMBED), jnp.float32),
          pltpu.VMEM((CH, EMBED), jnp.float32),
          pltpu.VMEM((CH, EMBED), jnp.float32),
          pltpu.SemaphoreType.DMA,
      ],
  )
  def gather(uidx_hbm, midx_hbm, utab_hbm, mtab_hbm, uout_hbm, mout_hbm,
             uidx_v, midx_v, ubuf_v, mbuf_v, uo_v, mo_v, sem):
    wid = lax.axis_index("s") * nc + lax.axis_index("c")
    base = wid * b_per_w
    pltpu.sync_copy(uidx_hbm.at[pl.ds(base, b_per_w)], uidx_v)
    pltpu.sync_copy(midx_hbm.at[pl.ds(base, b_per_w)], midx_v)

    for c in range(n_ch):
      def issue(g, carry):
        uvec = (uidx_v[pl.ds(c * CH + g * 16, 16)] >> 3) * GROUP
        mvec = (midx_v[pl.ds(c * CH + g * 16, 16)] >> 3) * GROUP
        for k in range(16):
          rr = pl.multiple_of(uvec[k], GROUP)
          pltpu.async_copy(utab_hbm.at[pl.ds(rr, GROUP)],
                           ubuf_v.at[pl.ds((g * 16 + k) * GROUP, GROUP)], sem)
          ss = pl.multiple_of(mvec[k], GROUP)
          pltpu.async_copy(mtab_hbm.at[pl.ds(ss, GROUP)],
                           mbuf_v.at[pl.ds((g * 16 + k) * GROUP, GROUP)], sem)
        return carry

      lax.fori_loop(0, CH // 16, issue, 0)
      pltpu.make_async_copy(uout_hbm.at[pl.ds(0, CH * GROUP)], ubuf_v,
                            sem).wait()
      pltpu.make_async_copy(mout_hbm.at[pl.ds(0, CH * GROUP)], mbuf_v,
                            sem).wait()

      def select(g, carry):
        uq = (uidx_v[pl.ds(c * CH + g * 16, 16)] & (GROUP - 1))
        mq = (midx_v[pl.ds(c * CH + g * 16, 16)] & (GROUP - 1))
        for k in range(16):
          i = g * 16 + k
          ur = i * GROUP + uq[k]
          uo_v[i, pl.ds(0, 16)] = ubuf_v[ur, pl.ds(0, 16)]
          uo_v[i, pl.ds(16, 16)] = ubuf_v[ur, pl.ds(16, 16)]
          mr = i * GROUP + mq[k]
          mo_v[i, pl.ds(0, 16)] = mbuf_v[mr, pl.ds(0, 16)]
          mo_v[i, pl.ds(16, 16)] = mbuf_v[mr, pl.ds(16, 16)]
        return carry

      lax.fori_loop(0, CH // 16, select, 0)
      pltpu.sync_copy(uo_v, uout_hbm.at[pl.ds(base + c * CH, CH)])
      pltpu.sync_copy(mo_v, mout_hbm.at[pl.ds(base + c * CH, CH)])

  return gather


def _mlp_body(u_ref, m_ref, w1_ref, b1_ref, w2_ref, b2_ref, w3_ref, b3_ref,
              o_ref):
  h1 = jnp.dot(u_ref[...], w1_ref[0:EMBED, :],
               preferred_element_type=jnp.float32)
  h1 = h1 + jnp.dot(m_ref[...], w1_ref[EMBED:2 * EMBED, :],
                    preferred_element_type=jnp.float32)
  h1 = jnp.maximum(h1 + b1_ref[...], 0.0)
  h2 = jnp.dot(h1, w2_ref[...], preferred_element_type=jnp.float32)
  h2 = jnp.maximum(h2 + b2_ref[...], 0.0)
  o_ref[...] = jnp.sum(h2 * w3_ref[...], axis=1, keepdims=True) + b3_ref[...]


def _mlp_call(u_emb, m_emb, W1, b1, W2, b2, W3, b3):
  bb = 2048
  grid = (BATCH // bb,)
  return pl.pallas_call(
      _mlp_body,
      grid=grid,
      in_specs=[
          pl.BlockSpec((bb, EMBED), lambda i: (i, 0)),
          pl.BlockSpec((bb, EMBED), lambda i: (i, 0)),
          pl.BlockSpec((2 * EMBED, 128), lambda i: (0, 0)),
          pl.BlockSpec((1, 128), lambda i: (0, 0)),
          pl.BlockSpec((128, 64), lambda i: (0, 0)),
          pl.BlockSpec((1, 64), lambda i: (0, 0)),
          pl.BlockSpec((1, 64), lambda i: (0, 0)),
          pl.BlockSpec((1, 1), lambda i: (0, 0)),
      ],
      out_specs=pl.BlockSpec((bb, 1), lambda i: (i, 0)),
      out_shape=jax.ShapeDtypeStruct((BATCH, 1), jnp.float32),
  )(u_emb, m_emb, W1, b1.reshape(1, 128), W2, b2.reshape(1, 64),
    W3.reshape(1, 64), b3.reshape(1, 1))


def kernel(user_input, movie_input, user_table, movie_table,
           W1, b1, W2, b2, W3, b3):
  utab = _transpose_call(user_table.T)
  mtab = _transpose_call(movie_table.T)
  gather = _make_gather()
  u_emb, m_emb = gather(user_input, movie_input, utab, mtab)
  return _mlp_call(u_emb, m_emb, W1, b1, W2, b2, W3, b3)
